# Initial kernel scaffold; baseline (speedup 1.0000x reference)
import jax
import jax.numpy as jnp
from jax.experimental import pallas as pl


def kernel(x, batch):
    raise NotImplementedError("write your pallas kernel here")


# speedup vs baseline: 1.0860x; 1.0860x over previous
import jax
import jax.numpy as jnp
from jax import lax
from jax.experimental import pallas as pl
from jax.experimental.pallas import tpu as pltpu
from jax.experimental.pallas import tpu_sc as plsc

N = 320000
D = 128
S = 10000
NC = 2
NS = 16
NW = NC * NS
ROWS_PER_W = N // NW
CHUNK = 80
NCHUNKS = ROWS_PER_W // CHUNK
W = 256
LOGF = 7
F = 128
MAXF = 82
FDIM = MAXF * F
S_BUF = FDIM
CW = 16
L = 16


def _sc_flush_sums(x, batch3d):
    mesh = plsc.VectorSubcoreMesh(core_axis_name="c", subcore_axis_name="s")

    def body(x_hbm, idx_hbm, pf_sum, pf_cnt, desc_hbm,
             idx_v, xbuf, acc, cnt, desc_v):
        iota = lax.iota(jnp.int32, L)
        cinc = jnp.full((L,), 0.0625, jnp.float32)
        zf32 = jnp.zeros((L,), jnp.float32)
        zi32 = jnp.zeros((L,), jnp.int32)
        c = lax.axis_index("c")
        s = lax.axis_index("s")
        wid = c * NS + s

        def zrow(r, carry):
            for u in range(D // L):
                acc[r, pl.ds(u * L, L)] = zf32
            cnt[r, pl.ds(0, L)] = zf32
            return carry

        lax.fori_loop(0, W, zrow, 0)

        pltpu.sync_copy(idx_hbm.at[wid], idx_v)

        first = idx_v[0, pl.ds(0, L)][0]
        k0 = first >> LOGF
        b0 = k0 << LOGF

        row_base = wid * ROWS_PER_W

        def flush(slot, base):
            loc = (base - b0) & (W - 1)
            pltpu.sync_copy(acc.at[pl.ds(loc, F)],
                            pf_sum.at[wid, pl.ds(slot * F, F)])
            pltpu.sync_copy(cnt.at[pl.ds(loc, F)],
                            pf_cnt.at[wid, pl.ds(slot * F, F)])

        def flush_and_clear(t, carry):
            b, nf = carry
            flush(nf, b)
            loc = (b - b0) & (W - 1)

            def zr(r, carry2):
                for u in range(D // L):
                    acc[loc + r, pl.ds(u * L, L)] = zf32
                cnt[loc + r, pl.ds(0, L)] = zf32
                return carry2

            lax.fori_loop(0, F, zr, 0)
            return (b + F, nf + 1)

        def chunk_body(j, carry):
            pltpu.sync_copy(x_hbm.at[pl.ds(row_base + j * CHUNK, CHUNK)],
                            xbuf)
            b2, nf2 = carry
            for g in range(CHUNK // L):
                idvec = idx_v[j, pl.ds(g * L, L)]
                for rr in range(L):
                    r = g * L + rr
                    sid = idvec[rr]
                    need = jnp.maximum((sid - (b2 + W) + F) >> LOGF, 0)
                    b2, nf2 = lax.fori_loop(0, need, flush_and_clear,
                                            (b2, nf2))
                    lrow = (sid - b0) & (W - 1)
                    for u in range(D // L):
                        plsc.addupdate(acc.at[lrow, pl.ds(u * L, L)],
                                       xbuf[r, pl.ds(u * L, L)])
                    plsc.addupdate(cnt.at[lrow, pl.ds(0, L)], cinc)
            return (b2, nf2)

        b_fin, nf_fin = lax.fori_loop(0, NCHUNKS, chunk_body, (b0, 0))

        flush(nf_fin, b_fin)
        flush(nf_fin + 1, b_fin + F)

        dvec = jnp.where(iota == 0, k0,
                         jnp.where(iota == 1, nf_fin + 2, 0))
        desc_v[0, pl.ds(0, L)] = dvec
        pltpu.sync_copy(desc_v, desc_hbm.at[wid])

    run = pl.kernel(
        body,
        out_type=(
            jax.ShapeDtypeStruct((NW, FDIM, D), jnp.float32),
            jax.ShapeDtypeStruct((NW, FDIM, CW), jnp.float32),
            jax.ShapeDtypeStruct((NW, 1, L), jnp.int32),
        ),
        mesh=mesh,
        scratch_types=[
            pltpu.VMEM((NCHUNKS, CHUNK), jnp.int32),
            pltpu.VMEM((CHUNK, D), jnp.float32),
            pltpu.VMEM((W, D), jnp.float32),
            pltpu.VMEM((W, CW), jnp.float32),
            pltpu.VMEM((1, L), jnp.int32),
        ],
    )
    return run(x, batch3d)


def _combine(pf_sum, pf_cnt, desc):
    def body(desc_ref, pfs_ref, pfc_ref, out_ref,
             acc_scr, cnt_scr, fbuf, fcbuf, sem1, sem2):
        acc_scr[...] = jnp.zeros((S_BUF, D), jnp.float32)
        cnt_scr[...] = jnp.zeros((S_BUF, CW), jnp.float32)

        for w in range(NW):
            k0 = desc_ref[w, 0, 0]
            nf = desc_ref[w, 0, 1]

            def fb(f, carry):
                cp1 = pltpu.make_async_copy(
                    pfs_ref.at[w, pl.ds(f * F, F)], fbuf, sem1)
                cp2 = pltpu.make_async_copy(
                    pfc_ref.at[w, pl.ds(f * F, F)], fcbuf, sem2)
                cp1.start()
                cp2.start()
                cp1.wait()
                cp2.wait()
                base = (k0 + f) * F
                acc_scr[pl.ds(base, F), :] += fbuf[...]
                cnt_scr[pl.ds(base, F), :] += fcbuf[...]
                return carry

            lax.fori_loop(0, nf, fb, 0)

        counts = jnp.sum(cnt_scr[0:S, :], axis=1, keepdims=True)
        out_ref[...] = acc_scr[0:S, :] / jnp.maximum(counts, 1.0)

    return pl.pallas_call(
        body,
        grid=(1,),
        in_specs=[
            pl.BlockSpec(memory_space=pltpu.SMEM),
            pl.BlockSpec(memory_space=pl.ANY),
            pl.BlockSpec(memory_space=pl.ANY),
        ],
        out_specs=pl.BlockSpec((S, D), lambda i: (0, 0)),
        out_shape=jax.ShapeDtypeStruct((S, D), jnp.float32),
        scratch_shapes=[
            pltpu.VMEM((S_BUF, D), jnp.float32),
            pltpu.VMEM((S_BUF, CW), jnp.float32),
            pltpu.VMEM((F, D), jnp.float32),
            pltpu.VMEM((F, CW), jnp.float32),
            pltpu.SemaphoreType.DMA,
            pltpu.SemaphoreType.DMA,
        ],
    )(desc, pf_sum, pf_cnt)


def kernel(x, batch):
    batch3d = batch.reshape(NW, NCHUNKS, CHUNK)
    pf_sum, pf_cnt, desc = _sc_flush_sums(x, batch3d)
    return _combine(pf_sum, pf_cnt, desc)


# speedup vs baseline: 1.2796x; 1.1782x over previous
import jax
import jax.numpy as jnp
from jax import lax
from jax.experimental import pallas as pl
from jax.experimental.pallas import tpu as pltpu
from jax.experimental.pallas import tpu_sc as plsc

N = 320000
D = 128
S = 10000
NC = 2
NS = 16
NW = NC * NS
ROWS_PER_W = N // NW
CHUNK = 80
NCHUNKS = ROWS_PER_W // CHUNK
W = 256
LOGF = 7
F = 128
MAXF = 82
FDIM = MAXF * F
S_BUF = FDIM
CW = 16
L = 16


def _sc_flush_sums(x, batch3d):
    mesh = plsc.VectorSubcoreMesh(core_axis_name="c", subcore_axis_name="s")

    def body(x_hbm, idx_hbm, pf_sum, pf_cnt, desc_hbm,
             idx_v, xbuf, acc, cnt, desc_v, xsem):
        iota = lax.iota(jnp.int32, L)
        cinc = jnp.full((L,), 0.0625, jnp.float32)
        zf32 = jnp.zeros((L,), jnp.float32)
        zi32 = jnp.zeros((L,), jnp.int32)
        c = lax.axis_index("c")
        s = lax.axis_index("s")
        wid = c * NS + s

        def zrow(r, carry):
            for u in range(D // L):
                acc[r, pl.ds(u * L, L)] = zf32
            cnt[r, pl.ds(0, L)] = zf32
            return carry

        lax.fori_loop(0, W, zrow, 0)

        pltpu.sync_copy(idx_hbm.at[wid], idx_v)

        first = idx_v[0, pl.ds(0, L)][0]
        k0 = first >> LOGF
        b0 = k0 << LOGF

        row_base = wid * ROWS_PER_W

        def flush(slot, base):
            loc = (base - b0) & (W - 1)
            pltpu.sync_copy(acc.at[pl.ds(loc, F)],
                            pf_sum.at[wid, pl.ds(slot * F, F)])
            pltpu.sync_copy(cnt.at[pl.ds(loc, F)],
                            pf_cnt.at[wid, pl.ds(slot * F, F)])

        def flush_and_clear(t, carry):
            b, nf = carry
            flush(nf, b)
            loc = (b - b0) & (W - 1)

            def zr(r, carry2):
                for u in range(D // L):
                    acc[loc + r, pl.ds(u * L, L)] = zf32
                cnt[loc + r, pl.ds(0, L)] = zf32
                return carry2

            lax.fori_loop(0, F, zr, 0)
            return (b + F, nf + 1)

        def fetch(j):
            return pltpu.make_async_copy(
                x_hbm.at[pl.ds(row_base + j * CHUNK, CHUNK)],
                xbuf.at[j & 1], xsem)

        fetch(0).start()

        def chunk_body(j, carry):
            fetch(j).wait()

            @pl.when(j + 1 < NCHUNKS)
            def _():
                fetch(j + 1).start()

            jb = j & 1
            b2, nf2 = carry
            for g in range(CHUNK // L):
                idvec = idx_v[j, pl.ds(g * L, L)]
                for rr in range(L):
                    r = g * L + rr
                    sid = idvec[rr]
                    need = jnp.maximum((sid - (b2 + W) + F) >> LOGF, 0)
                    b2, nf2 = lax.fori_loop(0, need, flush_and_clear,
                                            (b2, nf2))
                    lrow = (sid - b0) & (W - 1)
                    for u in range(D // L):
                        plsc.addupdate(acc.at[lrow, pl.ds(u * L, L)],
                                       xbuf[jb, r, pl.ds(u * L, L)])
                    plsc.addupdate(cnt.at[lrow, pl.ds(0, L)], cinc)
            return (b2, nf2)

        b_fin, nf_fin = lax.fori_loop(0, NCHUNKS, chunk_body, (b0, 0))

        flush(nf_fin, b_fin)
        flush(nf_fin + 1, b_fin + F)

        dvec = jnp.where(iota == 0, k0,
                         jnp.where(iota == 1, nf_fin + 2, 0))
        desc_v[0, pl.ds(0, L)] = dvec
        pltpu.sync_copy(desc_v, desc_hbm.at[wid])

    run = pl.kernel(
        body,
        out_type=(
            jax.ShapeDtypeStruct((NW, FDIM, D), jnp.float32),
            jax.ShapeDtypeStruct((NW, FDIM, CW), jnp.float32),
            jax.ShapeDtypeStruct((NW, 1, L), jnp.int32),
        ),
        mesh=mesh,
        scratch_types=[
            pltpu.VMEM((NCHUNKS, CHUNK), jnp.int32),
            pltpu.VMEM((2, CHUNK, D), jnp.float32),
            pltpu.VMEM((W, D), jnp.float32),
            pltpu.VMEM((W, CW), jnp.float32),
            pltpu.VMEM((1, L), jnp.int32),
            pltpu.SemaphoreType.DMA,
        ],
    )
    return run(x, batch3d)


def _combine(pf_sum, pf_cnt, desc):
    def body(desc_ref, pfs_ref, pfc_ref, out_ref,
             acc_scr, cnt_scr, fbuf, fcbuf, sem1, sem2):
        acc_scr[...] = jnp.zeros((S_BUF, D), jnp.float32)
        cnt_scr[...] = jnp.zeros((S_BUF, CW), jnp.float32)

        for w in range(NW):
            k0 = desc_ref[w, 0, 0]
            nf = desc_ref[w, 0, 1]

            def fb(f, carry):
                cp1 = pltpu.make_async_copy(
                    pfs_ref.at[w, pl.ds(f * F, F)], fbuf, sem1)
                cp2 = pltpu.make_async_copy(
                    pfc_ref.at[w, pl.ds(f * F, F)], fcbuf, sem2)
                cp1.start()
                cp2.start()
                cp1.wait()
                cp2.wait()
                base = (k0 + f) * F
                acc_scr[pl.ds(base, F), :] += fbuf[...]
                cnt_scr[pl.ds(base, F), :] += fcbuf[...]
                return carry

            lax.fori_loop(0, nf, fb, 0)

        counts = jnp.sum(cnt_scr[0:S, :], axis=1, keepdims=True)
        out_ref[...] = acc_scr[0:S, :] / jnp.maximum(counts, 1.0)

    return pl.pallas_call(
        body,
        grid=(1,),
        in_specs=[
            pl.BlockSpec(memory_space=pltpu.SMEM),
            pl.BlockSpec(memory_space=pl.ANY),
            pl.BlockSpec(memory_space=pl.ANY),
        ],
        out_specs=pl.BlockSpec((S, D), lambda i: (0, 0)),
        out_shape=jax.ShapeDtypeStruct((S, D), jnp.float32),
        scratch_shapes=[
            pltpu.VMEM((S_BUF, D), jnp.float32),
            pltpu.VMEM((S_BUF, CW), jnp.float32),
            pltpu.VMEM((F, D), jnp.float32),
            pltpu.VMEM((F, CW), jnp.float32),
            pltpu.SemaphoreType.DMA,
            pltpu.SemaphoreType.DMA,
        ],
    )(desc, pf_sum, pf_cnt)


def kernel(x, batch):
    batch3d = batch.reshape(NW, NCHUNKS, CHUNK)
    pf_sum, pf_cnt, desc = _sc_flush_sums(x, batch3d)
    return _combine(pf_sum, pf_cnt, desc)
